# baseline (speedup 1.0000x reference)
import jax
import jax.numpy as jnp
from jax import lax
from jax.experimental import pallas as pl

N_ATOM_TYPES = 4
N_BOND_TYPES = 16

_GRID = 125
_NODE_B = 400
_EDGE_B = 6400


def _seg_sums(d, t, n_types):
    oh = (t[:, None] == lax.broadcasted_iota(jnp.int32, (1, n_types), 1)
          ).astype(jnp.float32)
    dn = (((0,), (0,)), ((), ()))
    a = lax.dot_general(oh, jnp.abs(d), dimension_numbers=dn,
                        preferred_element_type=jnp.float32)
    s = lax.dot_general(oh, d * d, dimension_numbers=dn,
                        preferred_element_type=jnp.float32)
    c = jnp.sum(oh, axis=0).reshape(1, n_types)
    return a, s, c


def _fused_body(nx_ref, nr_ref, nt_ref, ex_ref, er_ref, et_ref, nm_ref, em_ref,
                na_ref, ns_ref, nc_ref, ea_ref, es_ref, ec_ref, out_ref):
    i = pl.program_id(0)
    na, ns, nc = _seg_sums(nx_ref[...] - nr_ref[...], nt_ref[0, 0, :],
                           N_ATOM_TYPES)
    ea, es, ec = _seg_sums(ex_ref[...] - er_ref[...], et_ref[0, 0, :],
                           N_BOND_TYPES)

    @pl.when(i == 0)
    def _init():
        na_ref[...] = na
        ns_ref[...] = ns
        nc_ref[...] = nc
        ea_ref[...] = ea
        es_ref[...] = es
        ec_ref[...] = ec

    @pl.when(i > 0)
    def _acc():
        na_ref[...] += na
        ns_ref[...] += ns
        nc_ref[...] += nc
        ea_ref[...] += ea
        es_ref[...] += es
        ec_ref[...] += ec

    @pl.when(i == _GRID - 1)
    def _finish():
        def part(a, s, c, m):
            cc = jnp.maximum(c, 1.0)[:, None]
            mm = m * (c > 0.0).astype(jnp.float32)[:, None]
            denom = jnp.maximum(jnp.sum(mm), 1.0)
            mean_abs = jnp.sum((a / cc) * mm) / denom
            mean_sq = jnp.sum((s / cc) * mm) / denom
            return 0.5 * (mean_abs + jnp.sqrt(mean_sq))

        onsite = part(na_ref[...], ns_ref[...], nc_ref[0, :], nm_ref[...])
        hopping = part(ea_ref[...], es_ref[...], ec_ref[0, :], em_ref[...])
        out_ref[...] = (0.5 * (onsite + hopping))[None, None]


def kernel(node_features, ref_node_features, atom_type,
           edge_features, ref_edge_features, edge_type,
           mask_to_nrme, mask_to_erme):
    nt3 = atom_type.astype(jnp.int32).reshape(_GRID, 1, _NODE_B)
    et3 = edge_type.astype(jnp.int32).reshape(_GRID, 1, _EDGE_B)
    nw = node_features.shape[1]
    ew = edge_features.shape[1]
    outs = pl.pallas_call(
        _fused_body,
        grid=(_GRID,),
        in_specs=[
            pl.BlockSpec((_NODE_B, nw), lambda i: (i, 0)),
            pl.BlockSpec((_NODE_B, nw), lambda i: (i, 0)),
            pl.BlockSpec((1, 1, _NODE_B), lambda i: (i, 0, 0)),
            pl.BlockSpec((_EDGE_B, ew), lambda i: (i, 0)),
            pl.BlockSpec((_EDGE_B, ew), lambda i: (i, 0)),
            pl.BlockSpec((1, 1, _EDGE_B), lambda i: (i, 0, 0)),
            pl.BlockSpec((N_ATOM_TYPES, nw), lambda i: (0, 0)),
            pl.BlockSpec((N_BOND_TYPES, ew), lambda i: (0, 0)),
        ],
        out_specs=[
            pl.BlockSpec((N_ATOM_TYPES, nw), lambda i: (0, 0)),
            pl.BlockSpec((N_ATOM_TYPES, nw), lambda i: (0, 0)),
            pl.BlockSpec((1, N_ATOM_TYPES), lambda i: (0, 0)),
            pl.BlockSpec((N_BOND_TYPES, ew), lambda i: (0, 0)),
            pl.BlockSpec((N_BOND_TYPES, ew), lambda i: (0, 0)),
            pl.BlockSpec((1, N_BOND_TYPES), lambda i: (0, 0)),
            pl.BlockSpec((1, 1), lambda i: (0, 0)),
        ],
        out_shape=[
            jax.ShapeDtypeStruct((N_ATOM_TYPES, nw), jnp.float32),
            jax.ShapeDtypeStruct((N_ATOM_TYPES, nw), jnp.float32),
            jax.ShapeDtypeStruct((1, N_ATOM_TYPES), jnp.float32),
            jax.ShapeDtypeStruct((N_BOND_TYPES, ew), jnp.float32),
            jax.ShapeDtypeStruct((N_BOND_TYPES, ew), jnp.float32),
            jax.ShapeDtypeStruct((1, N_BOND_TYPES), jnp.float32),
            jax.ShapeDtypeStruct((1, 1), jnp.float32),
        ],
    )(node_features, ref_node_features, nt3,
      edge_features, ref_edge_features, et3,
      mask_to_nrme.astype(jnp.float32), mask_to_erme.astype(jnp.float32))
    return outs[6].reshape(())


# speedup vs baseline: 1.0084x; 1.0084x over previous
import jax
import jax.numpy as jnp
from jax import lax
from jax.experimental import pallas as pl

N_ATOM_TYPES = 4
N_BOND_TYPES = 16

_GRID = 125
_NODE_B = 400
_EDGE_B = 6400


def _seg_sums(d, t, n_types):
    oh = (t[:, None] == lax.broadcasted_iota(jnp.int32, (1, n_types), 1)
          ).astype(jnp.float32)
    dn = (((0,), (0,)), ((), ()))
    a = lax.dot_general(oh, jnp.abs(d), dimension_numbers=dn,
                        preferred_element_type=jnp.float32)
    s = lax.dot_general(oh, d * d, dimension_numbers=dn,
                        preferred_element_type=jnp.float32)
    c = jnp.sum(oh, axis=0).reshape(1, n_types)
    return a, s, c


def _fused_body(nx_ref, nr_ref, nt_ref, ex_ref, er_ref, et_ref,
                na_ref, ns_ref, nc_ref, ea_ref, es_ref, ec_ref):
    i = pl.program_id(0)
    na, ns, nc = _seg_sums(nx_ref[...] - nr_ref[...], nt_ref[0, 0, :],
                           N_ATOM_TYPES)
    ea, es, ec = _seg_sums(ex_ref[...] - er_ref[...], et_ref[0, 0, :],
                           N_BOND_TYPES)

    @pl.when(i == 0)
    def _init():
        na_ref[...] = na
        ns_ref[...] = ns
        nc_ref[...] = nc
        ea_ref[...] = ea
        es_ref[...] = es
        ec_ref[...] = ec

    @pl.when(i > 0)
    def _acc():
        na_ref[...] += na
        ns_ref[...] += ns
        nc_ref[...] += nc
        ea_ref[...] += ea
        es_ref[...] += es
        ec_ref[...] += ec


def _combine_body(na_ref, ns_ref, nc_ref, ea_ref, es_ref, ec_ref,
                  nm_ref, em_ref, out_ref):
    def part(a, s, c, m):
        cc = jnp.maximum(c, 1.0)[:, None]
        mm = m * (c > 0.0).astype(jnp.float32)[:, None]
        denom = jnp.maximum(jnp.sum(mm), 1.0)
        mean_abs = jnp.sum((a / cc) * mm) / denom
        mean_sq = jnp.sum((s / cc) * mm) / denom
        return 0.5 * (mean_abs + jnp.sqrt(mean_sq))

    onsite = part(na_ref[...], ns_ref[...], nc_ref[0, :], nm_ref[...])
    hopping = part(ea_ref[...], es_ref[...], ec_ref[0, :], em_ref[...])
    out_ref[...] = (0.5 * (onsite + hopping))[None, None]


def kernel(node_features, ref_node_features, atom_type,
           edge_features, ref_edge_features, edge_type,
           mask_to_nrme, mask_to_erme):
    nt3 = atom_type.astype(jnp.int32).reshape(_GRID, 1, _NODE_B)
    et3 = edge_type.astype(jnp.int32).reshape(_GRID, 1, _EDGE_B)
    nw = node_features.shape[1]
    ew = edge_features.shape[1]
    na, ns, nc, ea, es, ec = pl.pallas_call(
        _fused_body,
        grid=(_GRID,),
        in_specs=[
            pl.BlockSpec((_NODE_B, nw), lambda i: (i, 0)),
            pl.BlockSpec((_NODE_B, nw), lambda i: (i, 0)),
            pl.BlockSpec((1, 1, _NODE_B), lambda i: (i, 0, 0)),
            pl.BlockSpec((_EDGE_B, ew), lambda i: (i, 0)),
            pl.BlockSpec((_EDGE_B, ew), lambda i: (i, 0)),
            pl.BlockSpec((1, 1, _EDGE_B), lambda i: (i, 0, 0)),
        ],
        out_specs=[
            pl.BlockSpec((N_ATOM_TYPES, nw), lambda i: (0, 0)),
            pl.BlockSpec((N_ATOM_TYPES, nw), lambda i: (0, 0)),
            pl.BlockSpec((1, N_ATOM_TYPES), lambda i: (0, 0)),
            pl.BlockSpec((N_BOND_TYPES, ew), lambda i: (0, 0)),
            pl.BlockSpec((N_BOND_TYPES, ew), lambda i: (0, 0)),
            pl.BlockSpec((1, N_BOND_TYPES), lambda i: (0, 0)),
        ],
        out_shape=[
            jax.ShapeDtypeStruct((N_ATOM_TYPES, nw), jnp.float32),
            jax.ShapeDtypeStruct((N_ATOM_TYPES, nw), jnp.float32),
            jax.ShapeDtypeStruct((1, N_ATOM_TYPES), jnp.float32),
            jax.ShapeDtypeStruct((N_BOND_TYPES, ew), jnp.float32),
            jax.ShapeDtypeStruct((N_BOND_TYPES, ew), jnp.float32),
            jax.ShapeDtypeStruct((1, N_BOND_TYPES), jnp.float32),
        ],
    )(node_features, ref_node_features, nt3,
      edge_features, ref_edge_features, et3)
    out = pl.pallas_call(
        _combine_body,
        out_shape=jax.ShapeDtypeStruct((1, 1), jnp.float32),
    )(na, ns, nc, ea, es, ec,
      mask_to_nrme.astype(jnp.float32), mask_to_erme.astype(jnp.float32))
    return out.reshape(())
